# baseline (speedup 1.0000x reference)
import functools

import jax
import jax.numpy as jnp
import numpy as np
from jax.experimental import pallas as pl
from jax.experimental.pallas import tpu as pltpu

_B = 16384
_D = 128
_NUM_SPLITS = 16
_N_HEADS = 8
_ACTION_DIM = _NUM_SPLITS + 1
_NH = _N_HEADS * _ACTION_DIM
_TB = 1024


def _threefry2x32(k1, k2, x0, x1):
    ks0, ks1 = k1, k2
    ks2 = k1 ^ k2 ^ jnp.uint32(0x1BD11BDA)
    x0 = x0 + ks0
    x1 = x1 + ks1

    def rounds(x0, x1, rots):
        for r in rots:
            x0 = x0 + x1
            x1 = (x1 << jnp.uint32(r)) | (x1 >> jnp.uint32(32 - r))
            x1 = x0 ^ x1
        return x0, x1

    rot_a = (13, 15, 26, 6)
    rot_b = (17, 29, 16, 24)
    x0, x1 = rounds(x0, x1, rot_a)
    x0 = x0 + ks1
    x1 = x1 + ks2 + jnp.uint32(1)
    x0, x1 = rounds(x0, x1, rot_b)
    x0 = x0 + ks2
    x1 = x1 + ks0 + jnp.uint32(2)
    x0, x1 = rounds(x0, x1, rot_a)
    x0 = x0 + ks0
    x1 = x1 + ks1 + jnp.uint32(3)
    x0, x1 = rounds(x0, x1, rot_b)
    x0 = x0 + ks1
    x1 = x1 + ks2 + jnp.uint32(4)
    x0, x1 = rounds(x0, x1, rot_a)
    x0 = x0 + ks2
    x1 = x1 + ks0 + jnp.uint32(5)
    return x0, x1


def _gumbel_allheads():
    sample_key = jax.random.key(12345)
    keys = jax.vmap(jax.random.fold_in, (None, 0))(
        sample_key, jnp.arange(_N_HEADS, dtype=jnp.uint32))
    kd = jax.random.key_data(keys)
    k1 = jnp.repeat(kd[:, 0], _ACTION_DIM)[:, None]
    k2 = jnp.repeat(kd[:, 1], _ACTION_DIM)[:, None]
    c = (jnp.arange(_NH, dtype=jnp.uint32) % _ACTION_DIM)[:, None]
    r = jax.lax.broadcasted_iota(jnp.uint32, (_NH, _B), 1)
    cnt = r * jnp.uint32(_ACTION_DIM) + c
    b1, b2 = _threefry2x32(k1, k2, jnp.zeros_like(cnt), cnt)
    bits = b1 ^ b2
    fb = (bits >> jnp.uint32(9)) | jnp.uint32(0x3F800000)
    f = jax.lax.bitcast_convert_type(fb, jnp.float32) - jnp.float32(1.0)
    tiny = jnp.float32(np.finfo(np.float32).tiny)
    u = jnp.maximum(tiny, f * (jnp.float32(1.0) - tiny) + tiny)
    return -jnp.log(-jnp.log(u))


def _body(x_ref, w2_ref, b_ref, g_ref, act_ref, lp_ref):
    logits = jax.lax.dot_general(
        w2_ref[...], x_ref[...],
        dimension_numbers=(((1,), (1,)), ((), ())),
        preferred_element_type=jnp.float32) + b_ref[...]
    tb = logits.shape[1]
    iota_i = jax.lax.broadcasted_iota(jnp.int32, (_ACTION_DIM, tb), 0)
    iota = iota_i.astype(jnp.float32)
    taken = jnp.zeros((1, tb), jnp.float32)
    lp_sum = jnp.zeros((1, tb), jnp.float32)
    acts = []
    for idx in range(_N_HEADS):
        l = logits[idx * _ACTION_DIM:(idx + 1) * _ACTION_DIM, :]
        gi = g_ref[idx * _ACTION_DIM:(idx + 1) * _ACTION_DIM, :]
        mask = iota <= (jnp.float32(_NUM_SPLITS) - taken)
        ml = jnp.where(mask, l, jnp.float32(-1e10))
        y = gi + ml
        m = jnp.max(y, axis=0, keepdims=True)
        a = jnp.min(jnp.where(y == m, iota, jnp.float32(1e9)),
                    axis=0, keepdims=True)
        mm = jnp.max(ml, axis=0, keepdims=True)
        lse = jnp.log(jnp.sum(jnp.exp(ml - mm), axis=0, keepdims=True))
        ml_a = jnp.sum(jnp.where(iota == a, ml - mm, 0.0), axis=0,
                       keepdims=True)
        lp_sum = lp_sum + (ml_a - lse)
        taken = taken + a
        acts.append(a)
    act_ref[...] = jnp.concatenate(acts, axis=0).T
    lp_ref[...] = lp_sum.T


@functools.partial(jax.jit, static_argnames=())
def kernel(x, W, b):
    g = _gumbel_allheads()
    w2 = W.reshape(_NH, _D)
    b2 = b.reshape(_NH, 1)
    grid = (_B // _TB,)
    actions, lp = pl.pallas_call(
        _body,
        grid=grid,
        in_specs=[
            pl.BlockSpec((_TB, _D), lambda i: (i, 0)),
            pl.BlockSpec((_NH, _D), lambda i: (0, 0)),
            pl.BlockSpec((_NH, 1), lambda i: (0, 0)),
            pl.BlockSpec((_NH, _TB), lambda i: (0, i)),
        ],
        out_specs=[
            pl.BlockSpec((_TB, _N_HEADS), lambda i: (i, 0)),
            pl.BlockSpec((_TB, 1), lambda i: (i, 0)),
        ],
        out_shape=[
            jax.ShapeDtypeStruct((_B, _N_HEADS), jnp.float32),
            jax.ShapeDtypeStruct((_B, 1), jnp.float32),
        ],
        compiler_params=pltpu.CompilerParams(
            dimension_semantics=("arbitrary",),
        ),
    )(x, w2, b2, g)
    return actions, lp


# speedup vs baseline: 1.0633x; 1.0633x over previous
import functools

import jax
import jax.numpy as jnp
import numpy as np
from jax.experimental import pallas as pl
from jax.experimental.pallas import tpu as pltpu

_B = 16384
_D = 128
_NUM_SPLITS = 16
_N_HEADS = 8
_ACTION_DIM = _NUM_SPLITS + 1
_NH = _N_HEADS * _ACTION_DIM
_TB = 1024


def _threefry2x32(k1, k2, x0, x1):
    ks0, ks1 = k1, k2
    ks2 = k1 ^ k2 ^ jnp.uint32(0x1BD11BDA)
    x0 = x0 + ks0
    x1 = x1 + ks1

    def rounds(x0, x1, rots):
        for r in rots:
            x0 = x0 + x1
            x1 = (x1 << jnp.uint32(r)) | (x1 >> jnp.uint32(32 - r))
            x1 = x0 ^ x1
        return x0, x1

    rot_a = (13, 15, 26, 6)
    rot_b = (17, 29, 16, 24)
    x0, x1 = rounds(x0, x1, rot_a)
    x0 = x0 + ks1
    x1 = x1 + ks2 + jnp.uint32(1)
    x0, x1 = rounds(x0, x1, rot_b)
    x0 = x0 + ks2
    x1 = x1 + ks0 + jnp.uint32(2)
    x0, x1 = rounds(x0, x1, rot_a)
    x0 = x0 + ks0
    x1 = x1 + ks1 + jnp.uint32(3)
    x0, x1 = rounds(x0, x1, rot_b)
    x0 = x0 + ks1
    x1 = x1 + ks2 + jnp.uint32(4)
    x0, x1 = rounds(x0, x1, rot_a)
    x0 = x0 + ks2
    x1 = x1 + ks0 + jnp.uint32(5)
    return x0, x1


def _gumbel_tile(aux, t, tb):
    k1 = aux[:, 0:1]
    k2 = aux[:, 1:2]
    c = aux[:, 2:3]
    r = jax.lax.broadcasted_iota(jnp.uint32, (_NH, tb), 1)
    row = r + jnp.uint32(tb) * t.astype(jnp.uint32)
    cnt = row * jnp.uint32(_ACTION_DIM) + c
    b1, b2 = _threefry2x32(k1, k2, jnp.zeros_like(cnt), cnt)
    bits = b1 ^ b2
    fb = (bits >> jnp.uint32(9)) | jnp.uint32(0x3F800000)
    f = jax.lax.bitcast_convert_type(fb, jnp.float32) - jnp.float32(1.0)
    tiny = jnp.float32(np.finfo(np.float32).tiny)
    u = jnp.maximum(tiny, f * (jnp.float32(1.0) - tiny) + tiny)
    return -jnp.log(-jnp.log(u))


def _body(x_ref, w2_ref, b_ref, aux_ref, act_ref, lp_ref):
    logits = jax.lax.dot_general(
        w2_ref[...], x_ref[...],
        dimension_numbers=(((1,), (1,)), ((), ())),
        preferred_element_type=jnp.float32) + b_ref[...]
    tb = logits.shape[1]
    g = _gumbel_tile(aux_ref[...], pl.program_id(0), tb)
    iota_i = jax.lax.broadcasted_iota(jnp.int32, (_ACTION_DIM, tb), 0)
    iota = iota_i.astype(jnp.float32)
    taken = jnp.zeros((1, tb), jnp.float32)
    lp_sum = jnp.zeros((1, tb), jnp.float32)
    acts = []
    for idx in range(_N_HEADS):
        l = logits[idx * _ACTION_DIM:(idx + 1) * _ACTION_DIM, :]
        gi = g[idx * _ACTION_DIM:(idx + 1) * _ACTION_DIM, :]
        mask = iota <= (jnp.float32(_NUM_SPLITS) - taken)
        ml = jnp.where(mask, l, jnp.float32(-1e10))
        y = gi + ml
        m = jnp.max(y, axis=0, keepdims=True)
        a = jnp.min(jnp.where(y == m, iota, jnp.float32(1e9)),
                    axis=0, keepdims=True)
        mm = jnp.max(ml, axis=0, keepdims=True)
        lse = jnp.log(jnp.sum(jnp.exp(ml - mm), axis=0, keepdims=True))
        ml_a = jnp.sum(jnp.where(iota == a, ml - mm, 0.0), axis=0,
                       keepdims=True)
        lp_sum = lp_sum + (ml_a - lse)
        taken = taken + a
        acts.append(a)
    act_ref[...] = jnp.concatenate(acts, axis=0).T
    lp_ref[...] = lp_sum.T


@functools.partial(jax.jit, static_argnames=())
def kernel(x, W, b):
    sample_key = jax.random.key(12345)
    keys = jax.vmap(jax.random.fold_in, (None, 0))(
        sample_key, jnp.arange(_N_HEADS, dtype=jnp.uint32))
    kd = jax.random.key_data(keys)
    aux = jnp.stack([
        jnp.repeat(kd[:, 0], _ACTION_DIM),
        jnp.repeat(kd[:, 1], _ACTION_DIM),
        jnp.arange(_NH, dtype=jnp.uint32) % _ACTION_DIM,
        jnp.zeros((_NH,), jnp.uint32),
    ], axis=1)
    w2 = W.reshape(_NH, _D)
    b2 = b.reshape(_NH, 1)
    grid = (_B // _TB,)
    actions, lp = pl.pallas_call(
        _body,
        grid=grid,
        in_specs=[
            pl.BlockSpec((_TB, _D), lambda i: (i, 0)),
            pl.BlockSpec((_NH, _D), lambda i: (0, 0)),
            pl.BlockSpec((_NH, 1), lambda i: (0, 0)),
            pl.BlockSpec((_NH, 4), lambda i: (0, 0)),
        ],
        out_specs=[
            pl.BlockSpec((_TB, _N_HEADS), lambda i: (i, 0)),
            pl.BlockSpec((_TB, 1), lambda i: (i, 0)),
        ],
        out_shape=[
            jax.ShapeDtypeStruct((_B, _N_HEADS), jnp.float32),
            jax.ShapeDtypeStruct((_B, 1), jnp.float32),
        ],
        compiler_params=pltpu.CompilerParams(
            dimension_semantics=("arbitrary",),
        ),
    )(x, w2, b2, aux)
    return actions, lp
